# baseline (speedup 1.0000x reference)
import jax
import jax.numpy as jnp
from jax.experimental import pallas as pl
from jax.experimental.pallas import tpu as pltpu

E = 16
TOPK = 2
B = 128
D = 1024
F = 1024
NF = 2
FB = F // NF
NA = B * TOPK
CH = 16


def _moe_body(x_ref, ids_ref, es_ref, w1g_ref, w1u_ref, w1sg_ref, w1su_ref,
              w2_ref, w2s_ref, out_ref, cnt_out_ref,
              cnt_s, off_s, cur_s, tok_s, sc_s):
    e = pl.program_id(0)
    f = pl.program_id(1)

    @pl.when((e == 0) & (f == 0))
    def _routing():
        out_ref[...] = jnp.zeros_like(out_ref)

        def _zero(i, _):
            cnt_s[i] = 0
            return 0
        jax.lax.fori_loop(0, E, _zero, 0, unroll=True)

        def _count(i, _):
            ee = ids_ref[i // TOPK, i % TOPK]
            cnt_s[ee] = cnt_s[ee] + 1
            return 0
        jax.lax.fori_loop(0, NA, _count, 0)

        def _scan(i, acc):
            off_s[i] = acc
            cur_s[i] = acc
            return acc + cnt_s[i]
        jax.lax.fori_loop(0, E, _scan, 0, unroll=True)

        def _place(i, _):
            t = i // TOPK
            k = i % TOPK
            ee = ids_ref[t, k]
            p = cur_s[ee]
            tok_s[p] = t
            sc_s[p] = es_ref[t, k]
            cur_s[ee] = p + 1
            return 0
        jax.lax.fori_loop(0, NA, _place, 0)

    cnt = cnt_s[e]
    base = off_s[e]

    @pl.when(f == 0)
    def _():
        cnt_out_ref[e] = cnt

    xv = x_ref[...]
    w1g = w1g_ref[0]
    w1u = w1u_ref[0]
    w1sg = w1sg_ref[0]
    w1su = w1su_ref[0]
    w2 = w2_ref[0]
    w2s = w2s_ref[0]
    lane_iota = jax.lax.broadcasted_iota(jnp.int32, (CH, B), 1)

    nch = (cnt + CH - 1) // CH

    def _chunk(c, _):
        idx0 = base + c * CH
        toks = []
        ws = []
        for j in range(CH):
            p = jnp.minimum(idx0 + j, NA - 1)
            valid = (c * CH + j) < cnt
            toks.append(jnp.where(valid, tok_s[p], -1))
            ws.append(jnp.where(valid, sc_s[p], 0.0))
        tokv = jnp.stack(toks).reshape(CH, 1)
        wv = jnp.stack(ws).reshape(CH, 1)

        g1h = (tokv == lane_iota).astype(jnp.float32)
        xa = jnp.dot(g1h, xv, preferred_element_type=jnp.float32)
        gate = jnp.dot(xa, w1g, preferred_element_type=jnp.float32) * w1sg
        up = jnp.dot(xa, w1u, preferred_element_type=jnp.float32) * w1su
        h = gate * jax.nn.sigmoid(gate) * up
        part = jnp.dot(h, w2, preferred_element_type=jnp.float32) * w2s

        sw = g1h * wv
        out_ref[...] += jax.lax.dot_general(
            sw, part, (((0,), (0,)), ((), ())),
            preferred_element_type=jnp.float32)
        return 0

    jax.lax.fori_loop(0, nch, _chunk, 0)


@jax.jit
def kernel(x, expert_ids, smooth_scales, expert_scales, x_active_mask,
           gmm1_weight, gmm1_weight_scale, gmm2_weight, gmm2_weight_scale):
    del smooth_scales, x_active_mask
    w1s3 = gmm1_weight_scale.reshape(E, 1, 2 * F)
    w2s3 = gmm2_weight_scale.reshape(E, 1, D)

    out, counts = pl.pallas_call(
        _moe_body,
        grid=(E, NF),
        in_specs=[
            pl.BlockSpec((B, D), lambda e, f: (0, 0)),
            pl.BlockSpec(memory_space=pltpu.SMEM),
            pl.BlockSpec(memory_space=pltpu.SMEM),
            pl.BlockSpec((1, D, FB), lambda e, f: (e, 0, f)),
            pl.BlockSpec((1, D, FB), lambda e, f: (e, 0, f + NF)),
            pl.BlockSpec((1, 1, FB), lambda e, f: (e, 0, f)),
            pl.BlockSpec((1, 1, FB), lambda e, f: (e, 0, f + NF)),
            pl.BlockSpec((1, FB, D), lambda e, f: (e, f, 0)),
            pl.BlockSpec((1, 1, D), lambda e, f: (e, 0, 0)),
        ],
        out_specs=[
            pl.BlockSpec((B, D), lambda e, f: (0, 0)),
            pl.BlockSpec(memory_space=pltpu.SMEM),
        ],
        out_shape=[
            jax.ShapeDtypeStruct((B, D), jnp.float32),
            jax.ShapeDtypeStruct((E,), jnp.int32),
        ],
        scratch_shapes=[
            pltpu.SMEM((E,), jnp.int32),
            pltpu.SMEM((E,), jnp.int32),
            pltpu.SMEM((E,), jnp.int32),
            pltpu.SMEM((NA,), jnp.int32),
            pltpu.SMEM((NA,), jnp.float32),
        ],
        compiler_params=pltpu.CompilerParams(
            dimension_semantics=("arbitrary", "arbitrary"),
        ),
    )(x, expert_ids, expert_scales, gmm1_weight, gmm1_weight,
      w1s3, w1s3, gmm2_weight, w2s3)
    return out, counts


# speedup vs baseline: 1.2120x; 1.2120x over previous
import jax
import jax.numpy as jnp
from jax.experimental import pallas as pl
from jax.experimental.pallas import tpu as pltpu

E = 16
TOPK = 2
B = 128
D = 1024
F = 1024
NF = 1
FB = F // NF
NA = B * TOPK
CH = 16
PMAX = 512


def _moe_body(x_ref, idsf_ref, esf_ref, ids_ref, w1g_ref, w1u_ref,
              w1sg_ref, w1su_ref, w2_ref, w2s_ref, out_ref, cnt_out_ref,
              sg_ref, sw_ref, pall_ref, cnt_s, ab8_s):
    e = pl.program_id(0)
    f = pl.program_id(1)

    @pl.when((e == 0) & (f == 0))
    def _routing():
        def _zero(i, _):
            cnt_s[i] = 0
            return 0
        jax.lax.fori_loop(0, E, _zero, 0, unroll=True)

        def _count(i, _):
            ee = ids_ref[i // TOPK, i % TOPK]
            cnt_s[ee] = cnt_s[ee] + 1
            return 0
        jax.lax.fori_loop(0, NA, _count, 0)

        def _scan(i, acc):
            ab8_s[i] = acc
            c = cnt_s[i]
            cnt_out_ref[i] = c
            return acc + (c + CH - 1) // CH
        jax.lax.fori_loop(0, E, _scan, 0, unroll=True)

        ef = idsf_ref[...]
        oh = (ef == jax.lax.broadcasted_iota(jnp.int32, (NA, E), 1)
              ).astype(jnp.float32)
        eqm = jnp.dot(oh, oh.T, preferred_element_type=jnp.float32)
        ltm = (jax.lax.broadcasted_iota(jnp.int32, (NA, NA), 1)
               < jax.lax.broadcasted_iota(jnp.int32, (NA, NA), 0)
               ).astype(jnp.float32)
        rank = jnp.sum(eqm * ltm, axis=1, keepdims=True)
        abv = jnp.stack([ab8_s[i] for i in range(E)]).reshape(E, 1
                        ).astype(jnp.float32)
        abmap = jnp.dot(oh, abv, preferred_element_type=jnp.float32)
        pvec = abmap * CH + rank
        pveci = pvec.astype(jnp.int32)
        pot = (pveci == jax.lax.broadcasted_iota(jnp.int32, (NA, PMAX), 1)
               ).astype(jnp.float32)
        tm = ((jax.lax.broadcasted_iota(jnp.int32, (NA, B), 0) // TOPK)
              == jax.lax.broadcasted_iota(jnp.int32, (NA, B), 1)
              ).astype(jnp.float32)
        dn = (((0,), (0,)), ((), ()))
        sg_ref[...] = jax.lax.dot_general(pot, tm, dn,
                                          preferred_element_type=jnp.float32)
        sw_ref[...] = jax.lax.dot_general(pot * esf_ref[...], tm, dn,
                                          preferred_element_type=jnp.float32)
        pall_ref[...] = jnp.zeros_like(pall_ref)

    cnt = cnt_s[e]
    ab8 = ab8_s[e]

    xv = x_ref[...]
    w1g = w1g_ref[0]
    w1u = w1u_ref[0]
    w1sg = w1sg_ref[0]
    w1su = w1su_ref[0]
    w2 = w2_ref[0]
    w2s = w2s_ref[0]

    nch = (cnt + CH - 1) // CH

    def _chunk(c, _):
        s = pl.multiple_of((ab8 + c) * CH, CH)
        g = sg_ref[pl.ds(s, CH), :]
        xa = jnp.dot(g, xv, preferred_element_type=jnp.float32)
        gate = jnp.dot(xa, w1g, preferred_element_type=jnp.float32) * w1sg
        up = jnp.dot(xa, w1u, preferred_element_type=jnp.float32) * w1su
        h = gate * jax.nn.sigmoid(gate) * up
        part = jnp.dot(h, w2, preferred_element_type=jnp.float32) * w2s

        @pl.when(f == 0)
        def _():
            pall_ref[pl.ds(s, CH), :] = part

        @pl.when(f != 0)
        def _():
            pall_ref[pl.ds(s, CH), :] += part
        return 0

    jax.lax.fori_loop(0, nch, _chunk, 0)

    @pl.when((e == E - 1) & (f == NF - 1))
    def _combine():
        out_ref[...] = jax.lax.dot_general(
            sw_ref[...], pall_ref[...], (((0,), (0,)), ((), ())),
            preferred_element_type=jnp.float32)


@jax.jit
def kernel(x, expert_ids, smooth_scales, expert_scales, x_active_mask,
           gmm1_weight, gmm1_weight_scale, gmm2_weight, gmm2_weight_scale):
    del smooth_scales, x_active_mask
    w1s3 = gmm1_weight_scale.reshape(E, 1, 2 * F)
    w2s3 = gmm2_weight_scale.reshape(E, 1, D)
    idsf = expert_ids.reshape(NA, 1)
    esf = expert_scales.reshape(NA, 1)

    out, counts = pl.pallas_call(
        _moe_body,
        grid=(E, NF),
        in_specs=[
            pl.BlockSpec((B, D), lambda e, f: (0, 0)),
            pl.BlockSpec((NA, 1), lambda e, f: (0, 0)),
            pl.BlockSpec((NA, 1), lambda e, f: (0, 0)),
            pl.BlockSpec(memory_space=pltpu.SMEM),
            pl.BlockSpec((1, D, FB), lambda e, f: (e, 0, f)),
            pl.BlockSpec((1, D, FB), lambda e, f: (e, 0, f + NF)),
            pl.BlockSpec((1, 1, FB), lambda e, f: (e, 0, f)),
            pl.BlockSpec((1, 1, FB), lambda e, f: (e, 0, f + NF)),
            pl.BlockSpec((1, FB, D), lambda e, f: (e, f, 0)),
            pl.BlockSpec((1, 1, D), lambda e, f: (e, 0, 0)),
        ],
        out_specs=[
            pl.BlockSpec((B, D), lambda e, f: (0, 0)),
            pl.BlockSpec(memory_space=pltpu.SMEM),
        ],
        out_shape=[
            jax.ShapeDtypeStruct((B, D), jnp.float32),
            jax.ShapeDtypeStruct((E,), jnp.int32),
        ],
        scratch_shapes=[
            pltpu.VMEM((PMAX, B), jnp.float32),
            pltpu.VMEM((PMAX, B), jnp.float32),
            pltpu.VMEM((PMAX, D), jnp.float32),
            pltpu.SMEM((E,), jnp.int32),
            pltpu.SMEM((E,), jnp.int32),
        ],
        compiler_params=pltpu.CompilerParams(
            dimension_semantics=("arbitrary", "arbitrary"),
        ),
    )(x, idsf, esf, expert_ids, gmm1_weight, gmm1_weight,
      w1s3, w1s3, gmm2_weight, w2s3)
    return out, counts


# speedup vs baseline: 1.2759x; 1.0527x over previous
import functools

import jax
import jax.numpy as jnp
from jax.experimental import pallas as pl
from jax.experimental.pallas import tpu as pltpu

E = 16
TOPK = 2
B = 128
D = 1024
F = 1024
NF = 2
FB = F // NF


def _moe_body(x_ref, ids_ref, es_ref, w1g_ref, w1u_ref, w1sg_ref, w1su_ref,
              w2_ref, w2s_ref, out_ref, cnt_ref):
    e = pl.program_id(0)
    f = pl.program_id(1)

    xv = x_ref[...].astype(jnp.bfloat16)
    w1g = w1g_ref[0].astype(jnp.bfloat16)
    w1u = w1u_ref[0].astype(jnp.bfloat16)
    gate = jnp.dot(xv, w1g, preferred_element_type=jnp.float32) * w1sg_ref[0]
    up = jnp.dot(xv, w1u, preferred_element_type=jnp.float32) * w1su_ref[0]
    h = gate * jax.nn.sigmoid(gate) * up
    y2 = jnp.dot(h.astype(jnp.bfloat16), w2_ref[0].astype(jnp.bfloat16),
                 preferred_element_type=jnp.float32) * w2s_ref[0]

    m = ids_ref[...] == e
    w = jnp.sum(jnp.where(m, es_ref[...], 0.0), axis=1, keepdims=True)
    contrib = w * y2

    first = (e == 0) & (f == 0)

    @pl.when(first)
    def _():
        out_ref[...] = contrib

    @pl.when(jnp.logical_not(first))
    def _():
        out_ref[...] += contrib

    @pl.when(f == 0)
    def _():
        cnt_ref[e] = jnp.sum(m.astype(jnp.int32))


@jax.jit
def kernel(x, expert_ids, smooth_scales, expert_scales, x_active_mask,
           gmm1_weight, gmm1_weight_scale, gmm2_weight, gmm2_weight_scale):
    del smooth_scales, x_active_mask
    w1s3 = gmm1_weight_scale.reshape(E, 1, 2 * F)
    w2s3 = gmm2_weight_scale.reshape(E, 1, D)

    out, counts = pl.pallas_call(
        _moe_body,
        grid=(E, NF),
        in_specs=[
            pl.BlockSpec((B, D), lambda e, f: (0, 0)),
            pl.BlockSpec((B, TOPK), lambda e, f: (0, 0)),
            pl.BlockSpec((B, TOPK), lambda e, f: (0, 0)),
            pl.BlockSpec((1, D, FB), lambda e, f: (e, 0, f)),
            pl.BlockSpec((1, D, FB), lambda e, f: (e, 0, f + NF)),
            pl.BlockSpec((1, 1, FB), lambda e, f: (e, 0, f)),
            pl.BlockSpec((1, 1, FB), lambda e, f: (e, 0, f + NF)),
            pl.BlockSpec((1, FB, D), lambda e, f: (e, f, 0)),
            pl.BlockSpec((1, 1, D), lambda e, f: (e, 0, 0)),
        ],
        out_specs=[
            pl.BlockSpec((B, D), lambda e, f: (0, 0)),
            pl.BlockSpec(memory_space=pltpu.SMEM),
        ],
        out_shape=[
            jax.ShapeDtypeStruct((B, D), jnp.float32),
            jax.ShapeDtypeStruct((E,), jnp.int32),
        ],
        compiler_params=pltpu.CompilerParams(
            dimension_semantics=("arbitrary", "arbitrary"),
        ),
    )(x, expert_ids, expert_scales, gmm1_weight, gmm1_weight,
      w1s3, w1s3, gmm2_weight, w2s3)
    return out, counts


# speedup vs baseline: 1.3627x; 1.0680x over previous
import functools

import jax
import jax.numpy as jnp
from jax.experimental import pallas as pl
from jax.experimental.pallas import tpu as pltpu

E = 16
TOPK = 2
B = 128
D = 1024
F = 1024
NF = 1
FB = F // NF


def _moe_body(x_ref, ids_ref, es_ref, w1g_ref, w1u_ref, w1sg_ref, w1su_ref,
              w2_ref, w2s_ref, out_ref, cnt_ref):
    e = pl.program_id(0)
    f = pl.program_id(1)

    xv = x_ref[...].astype(jnp.bfloat16)
    w1g = w1g_ref[0].astype(jnp.bfloat16)
    w1u = w1u_ref[0].astype(jnp.bfloat16)
    gate = jnp.dot(xv, w1g, preferred_element_type=jnp.float32) * w1sg_ref[0]
    up = jnp.dot(xv, w1u, preferred_element_type=jnp.float32) * w1su_ref[0]
    h = gate * jax.nn.sigmoid(gate) * up
    y2 = jnp.dot(h.astype(jnp.bfloat16), w2_ref[0].astype(jnp.bfloat16),
                 preferred_element_type=jnp.float32) * w2s_ref[0]

    m = ids_ref[...] == e
    w = jnp.sum(jnp.where(m, es_ref[...], 0.0), axis=1, keepdims=True)
    contrib = w * y2

    first = (e == 0) & (f == 0)

    @pl.when(first)
    def _():
        out_ref[...] = contrib

    @pl.when(jnp.logical_not(first))
    def _():
        out_ref[...] += contrib

    @pl.when(f == 0)
    def _():
        cnt_ref[e] = jnp.sum(m.astype(jnp.int32))


@jax.jit
def kernel(x, expert_ids, smooth_scales, expert_scales, x_active_mask,
           gmm1_weight, gmm1_weight_scale, gmm2_weight, gmm2_weight_scale):
    del smooth_scales, x_active_mask
    w1s3 = gmm1_weight_scale.reshape(E, 1, 2 * F)
    w2s3 = gmm2_weight_scale.reshape(E, 1, D)

    out, counts = pl.pallas_call(
        _moe_body,
        grid=(E, NF),
        in_specs=[
            pl.BlockSpec((B, D), lambda e, f: (0, 0)),
            pl.BlockSpec((B, TOPK), lambda e, f: (0, 0)),
            pl.BlockSpec((B, TOPK), lambda e, f: (0, 0)),
            pl.BlockSpec((1, D, FB), lambda e, f: (e, 0, f)),
            pl.BlockSpec((1, D, FB), lambda e, f: (e, 0, f + NF)),
            pl.BlockSpec((1, 1, FB), lambda e, f: (e, 0, f)),
            pl.BlockSpec((1, 1, FB), lambda e, f: (e, 0, f + NF)),
            pl.BlockSpec((1, FB, D), lambda e, f: (e, f, 0)),
            pl.BlockSpec((1, 1, D), lambda e, f: (e, 0, 0)),
        ],
        out_specs=[
            pl.BlockSpec((B, D), lambda e, f: (0, 0)),
            pl.BlockSpec(memory_space=pltpu.SMEM),
        ],
        out_shape=[
            jax.ShapeDtypeStruct((B, D), jnp.float32),
            jax.ShapeDtypeStruct((E,), jnp.int32),
        ],
        compiler_params=pltpu.CompilerParams(
            dimension_semantics=("arbitrary", "arbitrary"),
        ),
    )(x, expert_ids, expert_scales, gmm1_weight, gmm1_weight,
      w1s3, w1s3, gmm2_weight, w2s3)
    return out, counts
